# baseline (speedup 1.0000x reference)
import jax
import jax.numpy as jnp
import numpy as np
from jax.experimental import pallas as pl
from jax.experimental.pallas import tpu as pltpu

_N = 1_000_000
_D = 64
_K = 100
_CH = 200
_CPB = 125
_B = _CH * _CPB
_NB = _N // _B
_NCH = _NB * _CPB
_MR = 40
_SR = 104
_G = 4

_NEG = np.float32(-np.inf)
_IMAX = np.int32(2147483647)


def _ka_cmax_select(s_ref, w_ref, doc_ref, ids_ref, q_ref, qs_ref, cms_ref):
    i = pl.program_id(0)

    @pl.when(i == 0)
    def _init():
        cms_ref[...] = jnp.full((_MR, _CPB), _NEG, jnp.float32)
        vec = jnp.full((1, _D), s_ref[0], jnp.float32)
        qrow = jnp.dot(vec, w_ref[...],
                       preferred_element_type=jnp.float32)
        qs_ref[...] = qrow.T
        q_ref[...] = qrow.T

    sv = jnp.dot(doc_ref[...], qs_ref[...],
                 preferred_element_type=jnp.float32)
    cm = jnp.max(sv.reshape(_CPB, _CH, 1), axis=1)
    cms_ref[pl.ds(i, 1), :] = cm.T

    @pl.when(i == _NB - 1)
    def _select():
        cid = (jax.lax.broadcasted_iota(jnp.int32, (_MR, _CPB), 0) * _CPB
               + jax.lax.broadcasted_iota(jnp.int32, (_MR, _CPB), 1))

        def body(t, v):
            m = jnp.max(v)
            j = jnp.min(jnp.where(v == m, cid, _IMAX))
            ids_ref[t] = j
            return jnp.where(cid == j, _NEG, v)

        jax.lax.fori_loop(0, _K, body, cms_ref[...])


def _kb_gather_select(ids_ref, tk_ref, q_ref, d0_ref, d1_ref, d2_ref, d3_ref,
                      os_ref, oi_ref, sv_ref, si_ref):
    i = pl.program_id(0)

    @pl.when(i == 0)
    def _init():
        sv_ref[...] = jnp.full((_SR, _CH), _NEG, jnp.float32)

    lane = jax.lax.broadcasted_iota(jnp.int32, (1, _CH), 1)
    for g, d_ref in enumerate((d0_ref, d1_ref, d2_ref, d3_ref)):
        sv = jnp.dot(d_ref[...], q_ref[...],
                     preferred_element_type=jnp.float32)
        r = i * _G + g
        sv_ref[pl.ds(r, 1), :] = sv.T
        si_ref[pl.ds(r, 1), :] = ids_ref[r] * _CH + lane

    @pl.when(i == _K // _G - 1)
    def _select():
        idxs = si_ref[...]
        tk = tk_ref[0]

        def body(t, v):
            m = jnp.max(v)
            j = jnp.min(jnp.where(v == m, idxs, _IMAX))
            valid = t < tk
            os_ref[t] = jnp.where(valid, m, _NEG)
            oi_ref[t] = jnp.where(valid, j, jnp.int32(-1))
            return jnp.where(idxs == j, _NEG, v)

        jax.lax.fori_loop(0, _K, body, sv_ref[...])


def kernel(query_scalar, doc_embeddings, W, top_k):
    tk = jnp.full((1,), top_k, jnp.int32)

    ids, q = pl.pallas_call(
        _ka_cmax_select,
        grid=(_NB,),
        in_specs=[
            pl.BlockSpec(memory_space=pltpu.SMEM),
            pl.BlockSpec((_D, _D), lambda i: (0, 0)),
            pl.BlockSpec((_B, _D), lambda i: (i, 0)),
        ],
        out_specs=[
            pl.BlockSpec(memory_space=pltpu.SMEM),
            pl.BlockSpec((_D, 1), lambda i: (0, 0)),
        ],
        out_shape=[
            jax.ShapeDtypeStruct((_K,), jnp.int32),
            jax.ShapeDtypeStruct((_D, 1), jnp.float32),
        ],
        scratch_shapes=[
            pltpu.VMEM((_D, 1), jnp.float32),
            pltpu.VMEM((_MR, _CPB), jnp.float32),
        ],
        compiler_params=pltpu.CompilerParams(
            dimension_semantics=("arbitrary",)),
    )(query_scalar, W, doc_embeddings)

    top_s, top_i = pl.pallas_call(
        _kb_gather_select,
        grid_spec=pltpu.PrefetchScalarGridSpec(
            num_scalar_prefetch=2,
            grid=(_K // _G,),
            in_specs=[
                pl.BlockSpec((_D, 1), lambda i, ids, tk: (0, 0)),
                pl.BlockSpec((_CH, _D), lambda i, ids, tk: (ids[i * _G], 0)),
                pl.BlockSpec((_CH, _D),
                             lambda i, ids, tk: (ids[i * _G + 1], 0)),
                pl.BlockSpec((_CH, _D),
                             lambda i, ids, tk: (ids[i * _G + 2], 0)),
                pl.BlockSpec((_CH, _D),
                             lambda i, ids, tk: (ids[i * _G + 3], 0)),
            ],
            out_specs=[
                pl.BlockSpec(memory_space=pltpu.SMEM),
                pl.BlockSpec(memory_space=pltpu.SMEM),
            ],
            scratch_shapes=[
                pltpu.VMEM((_SR, _CH), jnp.float32),
                pltpu.VMEM((_SR, _CH), jnp.int32),
            ],
        ),
        out_shape=[
            jax.ShapeDtypeStruct((_K,), jnp.float32),
            jax.ShapeDtypeStruct((_K,), jnp.int32),
        ],
    )(ids, tk, q, doc_embeddings, doc_embeddings, doc_embeddings,
      doc_embeddings)
    return top_s, top_i


# speedup vs baseline: 1.0268x; 1.0268x over previous
import jax
import jax.numpy as jnp
import numpy as np
from jax.experimental import pallas as pl
from jax.experimental.pallas import tpu as pltpu

_N = 1_000_000
_D = 64
_K = 100
_CH = 200
_CPB = 100
_B = _CH * _CPB
_NB = _N // _B
_NCH = _NB * _CPB
_MR = 56
_SR = 104
_G = 4

_NEG = np.float32(-np.inf)
_IMAX = np.int32(2147483647)


def _ka_cmax_select(s_ref, w_ref, doc_ref, ids_ref, q_ref, qs_ref, cms_ref):
    i = pl.program_id(0)

    @pl.when(i == 0)
    def _init():
        cms_ref[...] = jnp.full((_MR, _CPB), _NEG, jnp.float32)
        vec = jnp.full((1, _D), s_ref[0], jnp.float32)
        qrow = jnp.dot(vec, w_ref[...],
                       preferred_element_type=jnp.float32)
        qs_ref[...] = qrow.T
        q_ref[...] = qrow.T

    sv = jnp.dot(doc_ref[...], qs_ref[...],
                 preferred_element_type=jnp.float32)
    cm = jnp.max(sv.reshape(_CPB, _CH, 1), axis=1)
    cms_ref[pl.ds(i, 1), :] = cm.T

    @pl.when(i == _NB - 1)
    def _select():
        cid = (jax.lax.broadcasted_iota(jnp.int32, (_MR, _CPB), 0) * _CPB
               + jax.lax.broadcasted_iota(jnp.int32, (_MR, _CPB), 1))

        def body(t, v):
            m = jnp.max(v)
            j = jnp.min(jnp.where(v == m, cid, _IMAX))
            ids_ref[t] = j
            return jnp.where(cid == j, _NEG, v)

        jax.lax.fori_loop(0, _K, body, cms_ref[...])


def _kb_gather_select(ids_ref, tk_ref, q_ref, d0_ref, d1_ref, d2_ref, d3_ref,
                      os_ref, oi_ref, sv_ref, si_ref):
    i = pl.program_id(0)

    @pl.when(i == 0)
    def _init():
        sv_ref[...] = jnp.full((_SR, _CH), _NEG, jnp.float32)

    lane = jax.lax.broadcasted_iota(jnp.int32, (1, _CH), 1)
    for g, d_ref in enumerate((d0_ref, d1_ref, d2_ref, d3_ref)):
        sv = jnp.dot(d_ref[...], q_ref[...],
                     preferred_element_type=jnp.float32)
        r = i * _G + g
        sv_ref[pl.ds(r, 1), :] = sv.T
        si_ref[pl.ds(r, 1), :] = ids_ref[r] * _CH + lane

    @pl.when(i == _K // _G - 1)
    def _select():
        idxs = si_ref[...]
        tk = tk_ref[0]

        def body(t, v):
            m = jnp.max(v)
            j = jnp.min(jnp.where(v == m, idxs, _IMAX))
            valid = t < tk
            os_ref[t] = jnp.where(valid, m, _NEG)
            oi_ref[t] = jnp.where(valid, j, jnp.int32(-1))
            return jnp.where(idxs == j, _NEG, v)

        jax.lax.fori_loop(0, _K, body, sv_ref[...])


def kernel(query_scalar, doc_embeddings, W, top_k):
    tk = jnp.full((1,), top_k, jnp.int32)

    ids, q = pl.pallas_call(
        _ka_cmax_select,
        grid=(_NB,),
        in_specs=[
            pl.BlockSpec(memory_space=pltpu.SMEM),
            pl.BlockSpec((_D, _D), lambda i: (0, 0)),
            pl.BlockSpec((_B, _D), lambda i: (i, 0)),
        ],
        out_specs=[
            pl.BlockSpec(memory_space=pltpu.SMEM),
            pl.BlockSpec((_D, 1), lambda i: (0, 0)),
        ],
        out_shape=[
            jax.ShapeDtypeStruct((_K,), jnp.int32),
            jax.ShapeDtypeStruct((_D, 1), jnp.float32),
        ],
        scratch_shapes=[
            pltpu.VMEM((_D, 1), jnp.float32),
            pltpu.VMEM((_MR, _CPB), jnp.float32),
        ],
        compiler_params=pltpu.CompilerParams(
            dimension_semantics=("arbitrary",)),
    )(query_scalar, W, doc_embeddings)

    top_s, top_i = pl.pallas_call(
        _kb_gather_select,
        grid_spec=pltpu.PrefetchScalarGridSpec(
            num_scalar_prefetch=2,
            grid=(_K // _G,),
            in_specs=[
                pl.BlockSpec((_D, 1), lambda i, ids, tk: (0, 0)),
                pl.BlockSpec((_CH, _D), lambda i, ids, tk: (ids[i * _G], 0)),
                pl.BlockSpec((_CH, _D),
                             lambda i, ids, tk: (ids[i * _G + 1], 0)),
                pl.BlockSpec((_CH, _D),
                             lambda i, ids, tk: (ids[i * _G + 2], 0)),
                pl.BlockSpec((_CH, _D),
                             lambda i, ids, tk: (ids[i * _G + 3], 0)),
            ],
            out_specs=[
                pl.BlockSpec(memory_space=pltpu.SMEM),
                pl.BlockSpec(memory_space=pltpu.SMEM),
            ],
            scratch_shapes=[
                pltpu.VMEM((_SR, _CH), jnp.float32),
                pltpu.VMEM((_SR, _CH), jnp.int32),
            ],
        ),
        out_shape=[
            jax.ShapeDtypeStruct((_K,), jnp.float32),
            jax.ShapeDtypeStruct((_K,), jnp.int32),
        ],
    )(ids, tk, q, doc_embeddings, doc_embeddings, doc_embeddings,
      doc_embeddings)
    return top_s, top_i
